# Initial kernel scaffold; baseline (speedup 1.0000x reference)
import jax
import jax.numpy as jnp
from jax.experimental import pallas as pl


def kernel(x, edge_index, batch, W1, b1, W2, b2, Wfc, bfc):
    raise NotImplementedError("write your pallas kernel here")


# speedup vs baseline: 8.7463x; 8.7463x over previous
import functools

import jax
import jax.numpy as jnp
from jax import lax
from jax.experimental import pallas as pl
from jax.experimental.pallas import tpu as pltpu
from jax.experimental.pallas import tpu_sc as plsc

N = 10000
E = 320000
F = 128
H = 256
HALF = H // 2
G = 128
B = 64

NB = 10
RB = N // NB
CH = 80
EPT = E // 16
EPT32 = E // 32
DW = 128

_mesh = plsc.VectorSubcoreMesh(core_axis_name="c", subcore_axis_name="s")


@functools.partial(
    pl.kernel,
    out_type=[
        jax.ShapeDtypeStruct((N, DW), jnp.float32),
        jax.ShapeDtypeStruct((N, DW), jnp.float32),
    ],
    mesh=_mesh,
    scratch_types=[
        pltpu.VMEM((CH,), jnp.int32),
        pltpu.VMEM((CH, DW), jnp.float32),
        pltpu.VMEM_SHARED((N, DW), jnp.float32),
        pltpu.SemaphoreType.DMA,
    ],
)
def _deg_kernel(dst_hbm, ones_hbm, zdeg_hbm, dega_hbm, degb_hbm,
                idx_v, ones_v, acc, sem):
    c = lax.axis_index("c")
    s = lax.axis_index("s")

    @pl.when(s < 10)
    def _():
        pltpu.sync_copy(zdeg_hbm.at[pl.ds(s * 1000, 1000)],
                        acc.at[pl.ds(s * 1000, 1000)])

    pltpu.sync_copy(ones_hbm, ones_v)
    plsc.subcore_barrier()

    def body(j, carry):
        base = c * (E // 2) + s * EPT32 + j * CH
        pltpu.sync_copy(dst_hbm.at[pl.ds(base, CH)], idx_v)
        pltpu.sync_copy(ones_v, acc.at[idx_v], add=True)
        return carry

    lax.fori_loop(0, EPT32 // CH, body, 0)
    plsc.subcore_barrier()

    @pl.when((s < 10) & (c == 0))
    def _():
        pltpu.sync_copy(acc.at[pl.ds(s * 1000, 1000)],
                        dega_hbm.at[pl.ds(s * 1000, 1000)])

    @pl.when((s < 10) & (c == 1))
    def _():
        pltpu.sync_copy(acc.at[pl.ds(s * 1000, 1000)],
                        degb_hbm.at[pl.ds(s * 1000, 1000)])


@functools.partial(
    pl.kernel,
    out_type=[
        jax.ShapeDtypeStruct((N, HALF), jnp.float32),
        jax.ShapeDtypeStruct((N, HALF), jnp.float32),
    ],
    mesh=_mesh,
    scratch_types=[
        pltpu.VMEM((CH,), jnp.int32),
        pltpu.VMEM((CH,), jnp.int32),
        pltpu.VMEM((CH, HALF), jnp.float32),
        pltpu.VMEM_SHARED((N, HALF), jnp.float32),
        pltpu.SemaphoreType.DMA,
    ],
)
def _agg_kernel(xwa_hbm, xwb_hbm, src_hbm, dst_hbm, z2d_hbm,
                outa_hbm, outb_hbm, idx_s, idx_d, rows, acc, sem):
    c = lax.axis_index("c")
    s = lax.axis_index("s")

    @pl.when(s < 10)
    def _():
        pltpu.sync_copy(z2d_hbm.at[pl.ds(s * 1000, 1000)],
                        acc.at[pl.ds(s * 1000, 1000)])

    plsc.subcore_barrier()

    def body(j, carry):
        base = s * EPT + j * CH
        pltpu.sync_copy(src_hbm.at[pl.ds(base, CH)], idx_s)
        pltpu.sync_copy(dst_hbm.at[pl.ds(base, CH)], idx_d)

        @pl.when(c == 0)
        def _():
            pltpu.async_copy(xwa_hbm.at[idx_s], rows, sem).wait()

        @pl.when(c == 1)
        def _():
            pltpu.async_copy(xwb_hbm.at[idx_s], rows, sem).wait()

        pltpu.sync_copy(rows, acc.at[idx_d], add=True)
        return carry

    lax.fori_loop(0, EPT // CH, body, 0)
    plsc.subcore_barrier()

    @pl.when((s < 10) & (c == 0))
    def _():
        pltpu.sync_copy(acc.at[pl.ds(s * 1000, 1000)],
                        outa_hbm.at[pl.ds(s * 1000, 1000)])

    @pl.when((s < 10) & (c == 1))
    def _():
        pltpu.sync_copy(acc.at[pl.ds(s * 1000, 1000)],
                        outb_hbm.at[pl.ds(s * 1000, 1000)])


def _dis_block(dega_ref, degb_ref):
    deg = 1.0 + dega_ref[:, 0:1] + degb_ref[:, 0:1]
    return lax.rsqrt(deg)


def _tc1_body(x_ref, w1_ref, dega_ref, degb_ref, outa_ref, outb_ref):
    dis = _dis_block(dega_ref, degb_ref)
    xw = jnp.dot(x_ref[...], w1_ref[...], preferred_element_type=jnp.float32)
    xw = dis * xw
    outa_ref[...] = xw[:, :HALF]
    outb_ref[...] = xw[:, HALF:]


def _tc2_body(sa_ref, sb_ref, xa_ref, xb_ref, dega_ref, degb_ref,
              b1_ref, w2_ref, outa_ref, outb_ref):
    dis = _dis_block(dega_ref, degb_ref)
    ha = jnp.maximum(dis * (sa_ref[...] + xa_ref[...]) + b1_ref[0:1, :HALF], 0.0)
    hb = jnp.maximum(dis * (sb_ref[...] + xb_ref[...]) + b1_ref[0:1, HALF:], 0.0)
    h = jnp.concatenate([ha, hb], axis=1)
    xw = dis * jnp.dot(h, w2_ref[...], preferred_element_type=jnp.float32)
    outa_ref[...] = xw[:, :HALF]
    outb_ref[...] = xw[:, HALF:]


def _tc3_body(sa_ref, sb_ref, xa_ref, xb_ref, dega_ref, degb_ref,
              b2_ref, batch_ref, pooled_ref, counts_ref):
    i = pl.program_id(0)
    dis = _dis_block(dega_ref, degb_ref)
    ha = jnp.maximum(dis * (sa_ref[...] + xa_ref[...]) + b2_ref[0:1, :HALF], 0.0)
    hb = jnp.maximum(dis * (sb_ref[...] + xb_ref[...]) + b2_ref[0:1, HALF:], 0.0)
    h = jnp.concatenate([ha, hb], axis=1)
    bblk = batch_ref[0, 0, :]
    seg = lax.broadcasted_iota(jnp.int32, (B, RB), 0)
    p = (seg == bblk[None, :]).astype(jnp.float32)

    @pl.when(i == 0)
    def _():
        pooled_ref[...] = jnp.zeros_like(pooled_ref)
        counts_ref[...] = jnp.zeros_like(counts_ref)

    pooled_ref[...] += jnp.dot(p, h, preferred_element_type=jnp.float32)
    counts_ref[...] += jnp.dot(
        p, jnp.ones((RB, G), jnp.float32), preferred_element_type=jnp.float32)


def _tc4_body(pooled_ref, counts_ref, wfc_ref, bfc_ref, out_ref):
    cnt = jnp.maximum(counts_ref[:, 0:1], 1.0)
    out = jnp.dot(pooled_ref[...], wfc_ref[...],
                  preferred_element_type=jnp.float32)
    out_ref[...] = out / cnt + bfc_ref[0:1, :]


def _rowspec(width):
    return pl.BlockSpec((RB, width), lambda i: (i, 0))


def _fullspec(shape):
    nd = len(shape)
    return pl.BlockSpec(shape, lambda *_: (0,) * nd)


def kernel(x, edge_index, batch, W1, b1, W2, b2, Wfc, bfc):
    src = edge_index[0]
    dst = edge_index[1]
    z2d = jnp.zeros((N, HALF), jnp.float32)
    ones2d = jnp.ones((CH, DW), jnp.float32)

    dega, degb = _deg_kernel(dst, ones2d, z2d)

    xw1a, xw1b = pl.pallas_call(
        _tc1_body,
        grid=(NB,),
        in_specs=[_rowspec(F), _fullspec((F, H)), _rowspec(DW), _rowspec(DW)],
        out_specs=[_rowspec(HALF), _rowspec(HALF)],
        out_shape=[jax.ShapeDtypeStruct((N, HALF), jnp.float32)] * 2,
    )(x, W1, dega, degb)

    s1a, s1b = _agg_kernel(xw1a, xw1b, src, dst, z2d)

    xw2a, xw2b = pl.pallas_call(
        _tc2_body,
        grid=(NB,),
        in_specs=[_rowspec(HALF)] * 4 + [_rowspec(DW)] * 2
        + [_fullspec((1, H)), _fullspec((H, H))],
        out_specs=[_rowspec(HALF), _rowspec(HALF)],
        out_shape=[jax.ShapeDtypeStruct((N, HALF), jnp.float32)] * 2,
    )(s1a, s1b, xw1a, xw1b, dega, degb, b1.reshape(1, H), W2)

    s2a, s2b = _agg_kernel(xw2a, xw2b, src, dst, z2d)

    batch3 = batch.reshape(NB, 1, RB)
    pooled, counts = pl.pallas_call(
        _tc3_body,
        grid=(NB,),
        in_specs=[_rowspec(HALF)] * 4 + [_rowspec(DW)] * 2
        + [_fullspec((1, H)), pl.BlockSpec((1, 1, RB), lambda i: (i, 0, 0))],
        out_specs=[_fullspec((B, H)), _fullspec((B, G))],
        out_shape=[jax.ShapeDtypeStruct((B, H), jnp.float32),
                   jax.ShapeDtypeStruct((B, G), jnp.float32)],
    )(s2a, s2b, xw2a, xw2b, dega, degb, b2.reshape(1, H), batch3)

    out = pl.pallas_call(
        _tc4_body,
        in_specs=[_fullspec((B, H)), _fullspec((B, G)),
                  _fullspec((H, G)), _fullspec((1, G))],
        out_specs=_fullspec((B, G)),
        out_shape=jax.ShapeDtypeStruct((B, G), jnp.float32),
    )(pooled, counts, Wfc, bfc.reshape(1, G))
    return out


# speedup vs baseline: 14.4328x; 1.6502x over previous
import functools

import jax
import jax.numpy as jnp
from jax import lax
from jax.experimental import pallas as pl
from jax.experimental.pallas import tpu as pltpu
from jax.experimental.pallas import tpu_sc as plsc

N = 10000
E = 320000
F = 128
H = 256
HALF = H // 2
G = 128
B = 64

NB = 10
RB = N // NB
CH = 80
EPT = E // 16
EPT32 = E // 32
DW = 128

_mesh = plsc.VectorSubcoreMesh(core_axis_name="c", subcore_axis_name="s")


@functools.partial(
    pl.kernel,
    out_type=[
        jax.ShapeDtypeStruct((N, DW), jnp.float32),
        jax.ShapeDtypeStruct((N, DW), jnp.float32),
    ],
    mesh=_mesh,
    scratch_types=[
        pltpu.VMEM((CH,), jnp.int32),
        pltpu.VMEM((CH,), jnp.int32),
        pltpu.VMEM((CH, DW), jnp.float32),
        pltpu.VMEM_SHARED((N, DW), jnp.float32),
        pltpu.SemaphoreType.DMA,
        pltpu.SemaphoreType.DMA,
    ],
)
def _deg_kernel(dst_hbm, ones_hbm, zdeg_hbm, dega_hbm, degb_hbm,
                idx0, idx1, ones_v, acc, sem0, sem1):
    c = lax.axis_index("c")
    s = lax.axis_index("s")

    @pl.when(s < 10)
    def _():
        pltpu.sync_copy(zdeg_hbm.at[pl.ds(s * 1000, 1000)],
                        acc.at[pl.ds(s * 1000, 1000)])

    pltpu.sync_copy(ones_hbm, ones_v)
    plsc.subcore_barrier()

    ebase = c * (E // 2) + s * EPT32

    pltpu.sync_copy(dst_hbm.at[pl.ds(ebase, CH)], idx0)
    pltpu.async_copy(ones_v, acc.at[idx0], sem0, add=True)

    def body(i, carry):
        b1 = ebase + (2 * i + 1) * CH
        b2 = ebase + (2 * i + 2) * CH
        pltpu.sync_copy(dst_hbm.at[pl.ds(b1, CH)], idx1)
        pltpu.async_copy(ones_v, acc.at[idx1], sem1, add=True)
        pltpu.make_async_copy(ones_v, acc.at[idx0], sem0).wait()
        pltpu.sync_copy(dst_hbm.at[pl.ds(b2, CH)], idx0)
        pltpu.async_copy(ones_v, acc.at[idx0], sem0, add=True)
        pltpu.make_async_copy(ones_v, acc.at[idx1], sem1).wait()
        return carry

    lax.fori_loop(0, (EPT32 // CH - 1) // 2, body, 0)
    pltpu.make_async_copy(ones_v, acc.at[idx0], sem0).wait()
    plsc.subcore_barrier()

    @pl.when((s < 10) & (c == 0))
    def _():
        pltpu.sync_copy(acc.at[pl.ds(s * 1000, 1000)],
                        dega_hbm.at[pl.ds(s * 1000, 1000)])

    @pl.when((s < 10) & (c == 1))
    def _():
        pltpu.sync_copy(acc.at[pl.ds(s * 1000, 1000)],
                        degb_hbm.at[pl.ds(s * 1000, 1000)])


@functools.partial(
    pl.kernel,
    out_type=[
        jax.ShapeDtypeStruct((N, HALF), jnp.float32),
        jax.ShapeDtypeStruct((N, HALF), jnp.float32),
    ],
    mesh=_mesh,
    scratch_types=[
        pltpu.VMEM((CH,), jnp.int32),
        pltpu.VMEM((CH,), jnp.int32),
        pltpu.VMEM((CH,), jnp.int32),
        pltpu.VMEM((CH,), jnp.int32),
        pltpu.VMEM((CH, HALF), jnp.float32),
        pltpu.VMEM((CH, HALF), jnp.float32),
        pltpu.VMEM_SHARED((N, HALF), jnp.float32),
        pltpu.SemaphoreType.DMA,
        pltpu.SemaphoreType.DMA,
    ],
)
def _agg_kernel(xwa_hbm, xwb_hbm, src_hbm, dst_hbm, z2d_hbm,
                outa_hbm, outb_hbm, idx_s0, idx_d0, idx_s1, idx_d1,
                rows0, rows1, acc, sem0, sem1):
    c = lax.axis_index("c")
    s = lax.axis_index("s")

    @pl.when(s < 10)
    def _():
        pltpu.sync_copy(z2d_hbm.at[pl.ds(s * 1000, 1000)],
                        acc.at[pl.ds(s * 1000, 1000)])

    plsc.subcore_barrier()
    ebase = s * EPT
    nch = EPT // CH

    def load_idx(b, isr, idr):
        pltpu.sync_copy(src_hbm.at[pl.ds(b, CH)], isr)
        pltpu.sync_copy(dst_hbm.at[pl.ds(b, CH)], idr)

    def fire(isr, rows, sem):
        @pl.when(c == 0)
        def _():
            pltpu.async_copy(xwa_hbm.at[isr], rows, sem)

        @pl.when(c == 1)
        def _():
            pltpu.async_copy(xwb_hbm.at[isr], rows, sem)

    def gwait(rows, sem):
        pltpu.make_async_copy(xwa_hbm.at[pl.ds(0, CH)], rows, sem).wait()

    load_idx(ebase, idx_s0, idx_d0)
    fire(idx_s0, rows0, sem0)

    def body(i, carry):
        load_idx(ebase + (2 * i + 1) * CH, idx_s1, idx_d1)
        fire(idx_s1, rows1, sem1)
        gwait(rows0, sem0)
        pltpu.sync_copy(rows0, acc.at[idx_d0], add=True)

        @pl.when(i < nch // 2 - 1)
        def _():
            load_idx(ebase + (2 * i + 2) * CH, idx_s0, idx_d0)
            fire(idx_s0, rows0, sem0)

        gwait(rows1, sem1)
        pltpu.sync_copy(rows1, acc.at[idx_d1], add=True)
        return carry

    lax.fori_loop(0, nch // 2, body, 0)
    plsc.subcore_barrier()

    @pl.when((s < 10) & (c == 0))
    def _():
        pltpu.sync_copy(acc.at[pl.ds(s * 1000, 1000)],
                        outa_hbm.at[pl.ds(s * 1000, 1000)])

    @pl.when((s < 10) & (c == 1))
    def _():
        pltpu.sync_copy(acc.at[pl.ds(s * 1000, 1000)],
                        outb_hbm.at[pl.ds(s * 1000, 1000)])


def _dis_block(dega_ref, degb_ref):
    deg = 1.0 + dega_ref[:, 0:1] + degb_ref[:, 0:1]
    return lax.rsqrt(deg)


def _tc1_body(x_ref, w1_ref, dega_ref, degb_ref, outa_ref, outb_ref):
    dis = _dis_block(dega_ref, degb_ref)
    xw = jnp.dot(x_ref[...], w1_ref[...], preferred_element_type=jnp.float32)
    xw = dis * xw
    outa_ref[...] = xw[:, :HALF]
    outb_ref[...] = xw[:, HALF:]


def _tc2_body(sa_ref, sb_ref, xa_ref, xb_ref, dega_ref, degb_ref,
              b1_ref, w2_ref, outa_ref, outb_ref):
    dis = _dis_block(dega_ref, degb_ref)
    ha = jnp.maximum(dis * (sa_ref[...] + xa_ref[...]) + b1_ref[0:1, :HALF], 0.0)
    hb = jnp.maximum(dis * (sb_ref[...] + xb_ref[...]) + b1_ref[0:1, HALF:], 0.0)
    h = jnp.concatenate([ha, hb], axis=1)
    xw = dis * jnp.dot(h, w2_ref[...], preferred_element_type=jnp.float32)
    outa_ref[...] = xw[:, :HALF]
    outb_ref[...] = xw[:, HALF:]


def _tc3_body(sa_ref, sb_ref, xa_ref, xb_ref, dega_ref, degb_ref,
              b2_ref, batch_ref, pooled_ref, counts_ref):
    i = pl.program_id(0)
    dis = _dis_block(dega_ref, degb_ref)
    ha = jnp.maximum(dis * (sa_ref[...] + xa_ref[...]) + b2_ref[0:1, :HALF], 0.0)
    hb = jnp.maximum(dis * (sb_ref[...] + xb_ref[...]) + b2_ref[0:1, HALF:], 0.0)
    h = jnp.concatenate([ha, hb], axis=1)
    bblk = batch_ref[0, 0, :]
    seg = lax.broadcasted_iota(jnp.int32, (B, RB), 0)
    p = (seg == bblk[None, :]).astype(jnp.float32)

    @pl.when(i == 0)
    def _():
        pooled_ref[...] = jnp.zeros_like(pooled_ref)
        counts_ref[...] = jnp.zeros_like(counts_ref)

    pooled_ref[...] += jnp.dot(p, h, preferred_element_type=jnp.float32)
    counts_ref[...] += jnp.dot(
        p, jnp.ones((RB, G), jnp.float32), preferred_element_type=jnp.float32)


def _tc4_body(pooled_ref, counts_ref, wfc_ref, bfc_ref, out_ref):
    cnt = jnp.maximum(counts_ref[:, 0:1], 1.0)
    out = jnp.dot(pooled_ref[...], wfc_ref[...],
                  preferred_element_type=jnp.float32)
    out_ref[...] = out / cnt + bfc_ref[0:1, :]


def _rowspec(width):
    return pl.BlockSpec((RB, width), lambda i: (i, 0))


def _fullspec(shape):
    nd = len(shape)
    return pl.BlockSpec(shape, lambda *_: (0,) * nd)


def kernel(x, edge_index, batch, W1, b1, W2, b2, Wfc, bfc):
    src = edge_index[0]
    dst = edge_index[1]
    z2d = jnp.zeros((N, HALF), jnp.float32)
    ones2d = jnp.ones((CH, DW), jnp.float32)

    dega, degb = _deg_kernel(dst, ones2d, z2d)

    xw1a, xw1b = pl.pallas_call(
        _tc1_body,
        grid=(NB,),
        in_specs=[_rowspec(F), _fullspec((F, H)), _rowspec(DW), _rowspec(DW)],
        out_specs=[_rowspec(HALF), _rowspec(HALF)],
        out_shape=[jax.ShapeDtypeStruct((N, HALF), jnp.float32)] * 2,
    )(x, W1, dega, degb)

    s1a, s1b = _agg_kernel(xw1a, xw1b, src, dst, z2d)

    xw2a, xw2b = pl.pallas_call(
        _tc2_body,
        grid=(NB,),
        in_specs=[_rowspec(HALF)] * 4 + [_rowspec(DW)] * 2
        + [_fullspec((1, H)), _fullspec((H, H))],
        out_specs=[_rowspec(HALF), _rowspec(HALF)],
        out_shape=[jax.ShapeDtypeStruct((N, HALF), jnp.float32)] * 2,
    )(s1a, s1b, xw1a, xw1b, dega, degb, b1.reshape(1, H), W2)

    s2a, s2b = _agg_kernel(xw2a, xw2b, src, dst, z2d)

    batch3 = batch.reshape(NB, 1, RB)
    pooled, counts = pl.pallas_call(
        _tc3_body,
        grid=(NB,),
        in_specs=[_rowspec(HALF)] * 4 + [_rowspec(DW)] * 2
        + [_fullspec((1, H)), pl.BlockSpec((1, 1, RB), lambda i: (i, 0, 0))],
        out_specs=[_fullspec((B, H)), _fullspec((B, G))],
        out_shape=[jax.ShapeDtypeStruct((B, H), jnp.float32),
                   jax.ShapeDtypeStruct((B, G), jnp.float32)],
    )(s2a, s2b, xw2a, xw2b, dega, degb, b2.reshape(1, H), batch3)

    out = pl.pallas_call(
        _tc4_body,
        in_specs=[_fullspec((B, H)), _fullspec((B, G)),
                  _fullspec((H, G)), _fullspec((1, G))],
        out_specs=_fullspec((B, G)),
        out_shape=jax.ShapeDtypeStruct((B, G), jnp.float32),
    )(pooled, counts, Wfc, bfc.reshape(1, G))
    return out


# speedup vs baseline: 22.4837x; 1.5578x over previous
import functools

import jax
import jax.numpy as jnp
from jax import lax
from jax.experimental import pallas as pl
from jax.experimental.pallas import tpu as pltpu
from jax.experimental.pallas import tpu_sc as plsc

N = 10000
E = 320000
F = 128
H = 256
HALF = H // 2
G = 128
B = 64

NB = 10
RB = N // NB
CH = 80
ACH = 128
TEDGE = 19968
NCHG = 6
GW = NCHG * ACH
NBODY = TEDGE // (2 * GW)
EPT32 = E // 32
DW = 128

_mesh = plsc.VectorSubcoreMesh(core_axis_name="c", subcore_axis_name="s")


@functools.partial(
    pl.kernel,
    out_type=[
        jax.ShapeDtypeStruct((N, DW), jnp.float32),
        jax.ShapeDtypeStruct((N, DW), jnp.float32),
    ],
    mesh=_mesh,
    scratch_types=[
        pltpu.VMEM((CH,), jnp.int32),
        pltpu.VMEM((CH,), jnp.int32),
        pltpu.VMEM((CH, DW), jnp.float32),
        pltpu.VMEM_SHARED((N, DW), jnp.float32),
        pltpu.SemaphoreType.DMA,
        pltpu.SemaphoreType.DMA,
    ],
)
def _deg_kernel(dst_hbm, ones_hbm, zdeg_hbm, dega_hbm, degb_hbm,
                idx0, idx1, ones_v, acc, sem0, sem1):
    c = lax.axis_index("c")
    s = lax.axis_index("s")

    @pl.when(s < 10)
    def _():
        pltpu.sync_copy(zdeg_hbm.at[pl.ds(s * 1000, 1000)],
                        acc.at[pl.ds(s * 1000, 1000)])

    pltpu.sync_copy(ones_hbm, ones_v)
    plsc.subcore_barrier()

    ebase = c * (E // 2) + s * EPT32

    pltpu.sync_copy(dst_hbm.at[pl.ds(ebase, CH)], idx0)
    pltpu.async_copy(ones_v, acc.at[idx0], sem0, add=True)

    def body(i, carry):
        b1 = ebase + (2 * i + 1) * CH
        b2 = ebase + (2 * i + 2) * CH
        pltpu.sync_copy(dst_hbm.at[pl.ds(b1, CH)], idx1)
        pltpu.async_copy(ones_v, acc.at[idx1], sem1, add=True)
        pltpu.make_async_copy(ones_v, acc.at[idx0], sem0).wait()
        pltpu.sync_copy(dst_hbm.at[pl.ds(b2, CH)], idx0)
        pltpu.async_copy(ones_v, acc.at[idx0], sem0, add=True)
        pltpu.make_async_copy(ones_v, acc.at[idx1], sem1).wait()
        return carry

    lax.fori_loop(0, (EPT32 // CH - 1) // 2, body, 0)
    pltpu.make_async_copy(ones_v, acc.at[idx0], sem0).wait()
    plsc.subcore_barrier()

    @pl.when((s < 10) & (c == 0))
    def _():
        pltpu.sync_copy(acc.at[pl.ds(s * 1000, 1000)],
                        dega_hbm.at[pl.ds(s * 1000, 1000)])

    @pl.when((s < 10) & (c == 1))
    def _():
        pltpu.sync_copy(acc.at[pl.ds(s * 1000, 1000)],
                        degb_hbm.at[pl.ds(s * 1000, 1000)])


@functools.partial(
    pl.kernel,
    out_type=[
        jax.ShapeDtypeStruct((N, HALF), jnp.float32),
        jax.ShapeDtypeStruct((N, HALF), jnp.float32),
    ],
    mesh=_mesh,
    scratch_types=[
        [pltpu.VMEM((GW,), jnp.int32)] * 2,
        [pltpu.VMEM((GW,), jnp.int32)] * 2,
        [pltpu.VMEM((ACH, HALF), jnp.float32)] * 2,
        [pltpu.VMEM((ACH,), jnp.int32)] * 2,
        [pltpu.VMEM((ACH,), jnp.int32)] * 2,
        pltpu.VMEM_SHARED((N, HALF), jnp.float32),
        [pltpu.SemaphoreType.DMA] * 2,
        [pltpu.SemaphoreType.DMA] * 2,
        [pltpu.SemaphoreType.DMA] * 2,
    ],
)
def _agg_kernel(xwa_hbm, xwb_hbm, src_hbm, dst_hbm, z2d_hbm,
                outa_hbm, outb_hbm, isg, idg, rows, idxd, idxs, acc, si, sg, ss):
    c = lax.axis_index("c")
    s = lax.axis_index("s")

    @pl.when(s < 10)
    def _():
        pltpu.sync_copy(z2d_hbm.at[pl.ds(s * 1000, 1000)],
                        acc.at[pl.ds(s * 1000, 1000)])

    plsc.subcore_barrier()
    tbase = s * TEDGE

    def fire_idx(g, b):
        gb = tbase + g * GW
        pltpu.async_copy(src_hbm.at[pl.ds(gb, GW)], isg[b], si[b])
        pltpu.async_copy(dst_hbm.at[pl.ds(gb, GW)], idg[b], si[b])

    def wait_idx(b):
        pltpu.make_async_copy(src_hbm.at[pl.ds(0, GW)], isg[b], si[b]).wait()
        pltpu.make_async_copy(src_hbm.at[pl.ds(0, GW)], idg[b], si[b]).wait()

    def fire_gather(idx_ref, rbuf, sem):
        @pl.when(c == 0)
        def _():
            pltpu.async_copy(xwa_hbm.at[idx_ref], rbuf, sem)

        @pl.when(c == 1)
        def _():
            pltpu.async_copy(xwb_hbm.at[idx_ref], rbuf, sem)

    def gwait(k):
        pltpu.make_async_copy(xwa_hbm.at[pl.ds(0, ACH)], rows[k], sg[k]).wait()

    def swait(k):
        pltpu.make_async_copy(rows[k], acc.at[idxd[k]], ss[k]).wait()

    def fire_scatter(k):
        pltpu.async_copy(rows[k], acc.at[idxd[k]], ss[k], add=True)

    def visit(m, b, p, guard_head, guard_tail):
        k = p % 2

        def work_head():
            swait(k)

        def work_tail():
            gwait(k ^ 1)
            fire_scatter(k ^ 1)

        if guard_head:
            @pl.when(m > 0)
            def _():
                work_head()
        else:
            work_head()

        for v in range(ACH // 16):
            idxd[k][pl.ds(v * 16, 16)] = idg[b][pl.ds(p * ACH + v * 16, 16)]
            idxs[k][pl.ds(v * 16, 16)] = isg[b][pl.ds(p * ACH + v * 16, 16)]
        fire_gather(idxs[k], rows[k], sg[k])

        if guard_tail:
            @pl.when(m > 0)
            def _():
                work_tail()
        else:
            work_tail()

    pltpu.sync_copy(src_hbm.at[pl.ds(tbase, GW)], isg[0])
    pltpu.sync_copy(dst_hbm.at[pl.ds(tbase, GW)], idg[0])

    def body(m, carry):
        @pl.when(m > 0)
        def _():
            wait_idx(0)

        for p in range(NCHG):
            visit(m, 0, p, guard_head=(p < 2), guard_tail=(p < 1))
            if p == 1:
                fire_idx(2 * m + 1, 1)
        wait_idx(1)
        for p in range(NCHG):
            visit(m, 1, p, guard_head=False, guard_tail=False)
            if p == 1:
                @pl.when(m < NBODY - 1)
                def _():
                    fire_idx(2 * m + 2, 0)

        return carry

    lax.fori_loop(0, NBODY, body, 0)
    gwait(1)
    fire_scatter(1)
    swait(0)
    swait(1)

    @pl.when(s < 4)
    def _():
        lb = 16 * TEDGE + s * ACH
        pltpu.sync_copy(src_hbm.at[pl.ds(lb, ACH)], idxd[0])
        pltpu.sync_copy(dst_hbm.at[pl.ds(lb, ACH)], idxd[1])
        fire_gather(idxd[0], rows[0], sg[0])
        gwait(0)
        pltpu.sync_copy(rows[0], acc.at[idxd[1]], add=True)

    plsc.subcore_barrier()

    @pl.when((s < 10) & (c == 0))
    def _():
        pltpu.sync_copy(acc.at[pl.ds(s * 1000, 1000)],
                        outa_hbm.at[pl.ds(s * 1000, 1000)])

    @pl.when((s < 10) & (c == 1))
    def _():
        pltpu.sync_copy(acc.at[pl.ds(s * 1000, 1000)],
                        outb_hbm.at[pl.ds(s * 1000, 1000)])


def _dis_block(dega_ref, degb_ref):
    deg = 1.0 + dega_ref[:, 0:1] + degb_ref[:, 0:1]
    return lax.rsqrt(deg)


def _tc1_body(x_ref, w1_ref, dega_ref, degb_ref, outa_ref, outb_ref):
    dis = _dis_block(dega_ref, degb_ref)
    xw = jnp.dot(x_ref[...], w1_ref[...], preferred_element_type=jnp.float32)
    xw = dis * xw
    outa_ref[...] = xw[:, :HALF]
    outb_ref[...] = xw[:, HALF:]


def _tc2_body(sa_ref, sb_ref, xa_ref, xb_ref, dega_ref, degb_ref,
              b1_ref, w2_ref, outa_ref, outb_ref):
    dis = _dis_block(dega_ref, degb_ref)
    ha = jnp.maximum(dis * (sa_ref[...] + xa_ref[...]) + b1_ref[0:1, :HALF], 0.0)
    hb = jnp.maximum(dis * (sb_ref[...] + xb_ref[...]) + b1_ref[0:1, HALF:], 0.0)
    h = jnp.concatenate([ha, hb], axis=1)
    xw = dis * jnp.dot(h, w2_ref[...], preferred_element_type=jnp.float32)
    outa_ref[...] = xw[:, :HALF]
    outb_ref[...] = xw[:, HALF:]


def _tc3_body(sa_ref, sb_ref, xa_ref, xb_ref, dega_ref, degb_ref,
              b2_ref, batch_ref, pooled_ref, counts_ref):
    i = pl.program_id(0)
    dis = _dis_block(dega_ref, degb_ref)
    ha = jnp.maximum(dis * (sa_ref[...] + xa_ref[...]) + b2_ref[0:1, :HALF], 0.0)
    hb = jnp.maximum(dis * (sb_ref[...] + xb_ref[...]) + b2_ref[0:1, HALF:], 0.0)
    h = jnp.concatenate([ha, hb], axis=1)
    bblk = batch_ref[0, 0, :]
    seg = lax.broadcasted_iota(jnp.int32, (B, RB), 0)
    p = (seg == bblk[None, :]).astype(jnp.float32)

    @pl.when(i == 0)
    def _():
        pooled_ref[...] = jnp.zeros_like(pooled_ref)
        counts_ref[...] = jnp.zeros_like(counts_ref)

    pooled_ref[...] += jnp.dot(p, h, preferred_element_type=jnp.float32)
    counts_ref[...] += jnp.dot(
        p, jnp.ones((RB, G), jnp.float32), preferred_element_type=jnp.float32)


def _tc4_body(pooled_ref, counts_ref, wfc_ref, bfc_ref, out_ref):
    cnt = jnp.maximum(counts_ref[:, 0:1], 1.0)
    out = jnp.dot(pooled_ref[...], wfc_ref[...],
                  preferred_element_type=jnp.float32)
    out_ref[...] = out / cnt + bfc_ref[0:1, :]


def _rowspec(width):
    return pl.BlockSpec((RB, width), lambda i: (i, 0))


def _fullspec(shape):
    nd = len(shape)
    return pl.BlockSpec(shape, lambda *_: (0,) * nd)


def kernel(x, edge_index, batch, W1, b1, W2, b2, Wfc, bfc):
    src = edge_index[0]
    dst = edge_index[1]
    z2d = jnp.zeros((N, HALF), jnp.float32)
    ones2d = jnp.ones((CH, DW), jnp.float32)

    dega, degb = _deg_kernel(dst, ones2d, z2d)

    xw1a, xw1b = pl.pallas_call(
        _tc1_body,
        grid=(NB,),
        in_specs=[_rowspec(F), _fullspec((F, H)), _rowspec(DW), _rowspec(DW)],
        out_specs=[_rowspec(HALF), _rowspec(HALF)],
        out_shape=[jax.ShapeDtypeStruct((N, HALF), jnp.float32)] * 2,
    )(x, W1, dega, degb)

    s1a, s1b = _agg_kernel(xw1a, xw1b, src, dst, z2d)

    xw2a, xw2b = pl.pallas_call(
        _tc2_body,
        grid=(NB,),
        in_specs=[_rowspec(HALF)] * 4 + [_rowspec(DW)] * 2
        + [_fullspec((1, H)), _fullspec((H, H))],
        out_specs=[_rowspec(HALF), _rowspec(HALF)],
        out_shape=[jax.ShapeDtypeStruct((N, HALF), jnp.float32)] * 2,
    )(s1a, s1b, xw1a, xw1b, dega, degb, b1.reshape(1, H), W2)

    s2a, s2b = _agg_kernel(xw2a, xw2b, src, dst, z2d)

    batch3 = batch.reshape(NB, 1, RB)
    pooled, counts = pl.pallas_call(
        _tc3_body,
        grid=(NB,),
        in_specs=[_rowspec(HALF)] * 4 + [_rowspec(DW)] * 2
        + [_fullspec((1, H)), pl.BlockSpec((1, 1, RB), lambda i: (i, 0, 0))],
        out_specs=[_fullspec((B, H)), _fullspec((B, G))],
        out_shape=[jax.ShapeDtypeStruct((B, H), jnp.float32),
                   jax.ShapeDtypeStruct((B, G), jnp.float32)],
    )(s2a, s2b, xw2a, xw2b, dega, degb, b2.reshape(1, H), batch3)

    out = pl.pallas_call(
        _tc4_body,
        in_specs=[_fullspec((B, H)), _fullspec((B, G)),
                  _fullspec((H, G)), _fullspec((1, G))],
        out_specs=_fullspec((B, G)),
        out_shape=jax.ShapeDtypeStruct((B, G), jnp.float32),
    )(pooled, counts, Wfc, bfc.reshape(1, G))
    return out


# speedup vs baseline: 22.5032x; 1.0009x over previous
import functools

import jax
import jax.numpy as jnp
from jax import lax
from jax.experimental import pallas as pl
from jax.experimental.pallas import tpu as pltpu
from jax.experimental.pallas import tpu_sc as plsc

N = 10000
E = 320000
F = 128
H = 256
HALF = H // 2
G = 128
B = 64

NB = 10
RB = N // NB
CH = 80
ACH = 128
TEDGE = 19968
NCHG = 6
GW = NCHG * ACH
NBODY = TEDGE // (2 * GW)
EPT32 = E // 32
DW = 128

_mesh = plsc.VectorSubcoreMesh(core_axis_name="c", subcore_axis_name="s")


@functools.partial(
    pl.kernel,
    out_type=[
        jax.ShapeDtypeStruct((N, DW), jnp.float32),
        jax.ShapeDtypeStruct((N, DW), jnp.float32),
    ],
    mesh=_mesh,
    scratch_types=[
        pltpu.VMEM((CH,), jnp.int32),
        pltpu.VMEM((CH,), jnp.int32),
        pltpu.VMEM((CH, DW), jnp.float32),
        pltpu.VMEM_SHARED((N, DW), jnp.float32),
        pltpu.SemaphoreType.DMA,
        pltpu.SemaphoreType.DMA,
    ],
)
def _deg_kernel(dst_hbm, ones_hbm, zdeg_hbm, dega_hbm, degb_hbm,
                idx0, idx1, ones_v, acc, sem0, sem1):
    c = lax.axis_index("c")
    s = lax.axis_index("s")

    @pl.when(s < 10)
    def _():
        pltpu.sync_copy(zdeg_hbm.at[pl.ds(s * 1000, 1000)],
                        acc.at[pl.ds(s * 1000, 1000)])

    pltpu.sync_copy(ones_hbm, ones_v)
    plsc.subcore_barrier()

    ebase = c * (E // 2) + s * EPT32

    pltpu.sync_copy(dst_hbm.at[pl.ds(ebase, CH)], idx0)
    pltpu.async_copy(ones_v, acc.at[idx0], sem0, add=True)

    def body(i, carry):
        b1 = ebase + (2 * i + 1) * CH
        b2 = ebase + (2 * i + 2) * CH
        pltpu.sync_copy(dst_hbm.at[pl.ds(b1, CH)], idx1)
        pltpu.async_copy(ones_v, acc.at[idx1], sem1, add=True)
        pltpu.make_async_copy(ones_v, acc.at[idx0], sem0).wait()
        pltpu.sync_copy(dst_hbm.at[pl.ds(b2, CH)], idx0)
        pltpu.async_copy(ones_v, acc.at[idx0], sem0, add=True)
        pltpu.make_async_copy(ones_v, acc.at[idx1], sem1).wait()
        return carry

    lax.fori_loop(0, (EPT32 // CH - 1) // 2, body, 0)
    pltpu.make_async_copy(ones_v, acc.at[idx0], sem0).wait()
    plsc.subcore_barrier()

    @pl.when((s < 10) & (c == 0))
    def _():
        pltpu.sync_copy(acc.at[pl.ds(s * 1000, 1000)],
                        dega_hbm.at[pl.ds(s * 1000, 1000)])

    @pl.when((s < 10) & (c == 1))
    def _():
        pltpu.sync_copy(acc.at[pl.ds(s * 1000, 1000)],
                        degb_hbm.at[pl.ds(s * 1000, 1000)])


@functools.partial(
    pl.kernel,
    out_type=[
        jax.ShapeDtypeStruct((N, HALF), jnp.float32),
        jax.ShapeDtypeStruct((N, HALF), jnp.float32),
    ],
    mesh=_mesh,
    scratch_types=[
        [pltpu.VMEM((GW,), jnp.int32)] * 2,
        [pltpu.VMEM((GW,), jnp.int32)] * 2,
        [pltpu.VMEM((ACH, HALF), jnp.float32)] * 2,
        [pltpu.VMEM((ACH,), jnp.int32)] * 2,
        [pltpu.VMEM((ACH,), jnp.int32)] * 2,
        pltpu.VMEM_SHARED((N, HALF), jnp.float32),
        [pltpu.SemaphoreType.DMA] * 2,
        [pltpu.SemaphoreType.DMA] * 2,
        [pltpu.SemaphoreType.DMA] * 2,
    ],
)
def _agg_kernel(xwa_hbm, xwb_hbm, src_hbm, dst_hbm, z2d_hbm,
                outa_hbm, outb_hbm, isg, idg, rows, idxd, idxs, acc, si, sg, ss):
    c = lax.axis_index("c")
    s = lax.axis_index("s")

    @pl.when(s < 10)
    def _():
        pltpu.sync_copy(z2d_hbm.at[pl.ds(s * 1000, 1000)],
                        acc.at[pl.ds(s * 1000, 1000)])

    plsc.subcore_barrier()
    tbase = s * TEDGE

    def fire_idx(g, b):
        gb = tbase + g * GW
        pltpu.async_copy(src_hbm.at[pl.ds(gb, GW)], isg[b], si[b])
        pltpu.async_copy(dst_hbm.at[pl.ds(gb, GW)], idg[b], si[b])

    def wait_idx(b):
        pltpu.make_async_copy(src_hbm.at[pl.ds(0, GW)], isg[b], si[b]).wait()
        pltpu.make_async_copy(src_hbm.at[pl.ds(0, GW)], idg[b], si[b]).wait()

    def fire_gather(idx_ref, rbuf, sem):
        @pl.when(c == 0)
        def _():
            pltpu.async_copy(xwa_hbm.at[idx_ref], rbuf, sem)

        @pl.when(c == 1)
        def _():
            pltpu.async_copy(xwb_hbm.at[idx_ref], rbuf, sem)

    def gwait(k):
        pltpu.make_async_copy(xwa_hbm.at[pl.ds(0, ACH)], rows[k], sg[k]).wait()

    def swait(k):
        pltpu.make_async_copy(rows[k], acc.at[idxd[k]], ss[k]).wait()

    def fire_scatter(k):
        pltpu.async_copy(rows[k], acc.at[idxd[k]], ss[k], add=True)

    def visit(m, b, p, guard_head, guard_tail):
        k = p % 2

        def work_head():
            swait(k)

        def work_tail():
            gwait(k ^ 1)
            fire_scatter(k ^ 1)

        if guard_head:
            @pl.when(m > 0)
            def _():
                work_head()
        else:
            work_head()

        for v in range(ACH // 16):
            idxd[k][pl.ds(v * 16, 16)] = idg[b][pl.ds(p * ACH + v * 16, 16)]
            idxs[k][pl.ds(v * 16, 16)] = isg[b][pl.ds(p * ACH + v * 16, 16)]
        fire_gather(idxs[k], rows[k], sg[k])

        if guard_tail:
            @pl.when(m > 0)
            def _():
                work_tail()
        else:
            work_tail()

    pltpu.sync_copy(src_hbm.at[pl.ds(tbase, GW)], isg[0])
    pltpu.sync_copy(dst_hbm.at[pl.ds(tbase, GW)], idg[0])

    def body(m, carry):
        @pl.when(m > 0)
        def _():
            wait_idx(0)

        for p in range(NCHG):
            visit(m, 0, p, guard_head=(p < 2), guard_tail=(p < 1))
            if p == 1:
                fire_idx(2 * m + 1, 1)
        wait_idx(1)
        for p in range(NCHG):
            visit(m, 1, p, guard_head=False, guard_tail=False)
            if p == 1:
                @pl.when(m < NBODY - 1)
                def _():
                    fire_idx(2 * m + 2, 0)

        return carry

    lax.fori_loop(0, NBODY, body, 0)
    gwait(1)
    fire_scatter(1)
    swait(0)
    swait(1)

    @pl.when(s < 4)
    def _():
        lb = 16 * TEDGE + s * ACH
        pltpu.sync_copy(src_hbm.at[pl.ds(lb, ACH)], idxd[0])
        pltpu.sync_copy(dst_hbm.at[pl.ds(lb, ACH)], idxd[1])
        fire_gather(idxd[0], rows[0], sg[0])
        gwait(0)
        pltpu.sync_copy(rows[0], acc.at[idxd[1]], add=True)

    plsc.subcore_barrier()

    @pl.when((s < 10) & (c == 0))
    def _():
        pltpu.sync_copy(acc.at[pl.ds(s * 1000, 1000)],
                        outa_hbm.at[pl.ds(s * 1000, 1000)])

    @pl.when((s < 10) & (c == 1))
    def _():
        pltpu.sync_copy(acc.at[pl.ds(s * 1000, 1000)],
                        outb_hbm.at[pl.ds(s * 1000, 1000)])


def _dis_block(dega_ref, degb_ref):
    deg = 1.0 + dega_ref[:, 0:1] + degb_ref[:, 0:1]
    return lax.rsqrt(deg)


def _tc1_body(x_ref, w1_ref, dega_ref, degb_ref, outa_ref, outb_ref):
    dis = _dis_block(dega_ref, degb_ref)
    xw = jnp.dot(x_ref[...], w1_ref[...], preferred_element_type=jnp.float32)
    xw = dis * xw
    outa_ref[...] = xw[:, :HALF]
    outb_ref[...] = xw[:, HALF:]


def _tc2_body(sa_ref, sb_ref, xa_ref, xb_ref, dega_ref, degb_ref,
              b1_ref, w2_ref, outa_ref, outb_ref):
    dis = _dis_block(dega_ref, degb_ref)
    ha = jnp.maximum(dis * (sa_ref[...] + xa_ref[...]) + b1_ref[0:1, :HALF], 0.0)
    hb = jnp.maximum(dis * (sb_ref[...] + xb_ref[...]) + b1_ref[0:1, HALF:], 0.0)
    h = jnp.concatenate([ha, hb], axis=1)
    xw = dis * jnp.dot(h, w2_ref[...], preferred_element_type=jnp.float32)
    outa_ref[...] = xw[:, :HALF]
    outb_ref[...] = xw[:, HALF:]


def _tc3_body(sa_ref, sb_ref, xa_ref, xb_ref, dega_ref, degb_ref,
              b2_ref, batch_ref, wfc_ref, bfc_ref,
              pooled_ref, counts_ref, out_ref):
    i = pl.program_id(0)
    dis = _dis_block(dega_ref, degb_ref)
    ha = jnp.maximum(dis * (sa_ref[...] + xa_ref[...]) + b2_ref[0:1, :HALF], 0.0)
    hb = jnp.maximum(dis * (sb_ref[...] + xb_ref[...]) + b2_ref[0:1, HALF:], 0.0)
    h = jnp.concatenate([ha, hb], axis=1)
    bblk = batch_ref[0, 0, :]
    seg = lax.broadcasted_iota(jnp.int32, (B, RB), 0)
    p = (seg == bblk[None, :]).astype(jnp.float32)

    @pl.when(i == 0)
    def _():
        pooled_ref[...] = jnp.zeros_like(pooled_ref)
        counts_ref[...] = jnp.zeros_like(counts_ref)

    pooled_ref[...] += jnp.dot(p, h, preferred_element_type=jnp.float32)
    counts_ref[...] += jnp.dot(
        p, jnp.ones((RB, G), jnp.float32), preferred_element_type=jnp.float32)

    @pl.when(i == NB - 1)
    def _():
        cnt = jnp.maximum(counts_ref[:, 0:1], 1.0)
        out = jnp.dot(pooled_ref[...], wfc_ref[...],
                      preferred_element_type=jnp.float32)
        out_ref[...] = out / cnt + bfc_ref[0:1, :]


def _rowspec(width):
    return pl.BlockSpec((RB, width), lambda i: (i, 0))


def _fullspec(shape):
    nd = len(shape)
    return pl.BlockSpec(shape, lambda *_: (0,) * nd)


def kernel(x, edge_index, batch, W1, b1, W2, b2, Wfc, bfc):
    src = edge_index[0]
    dst = edge_index[1]
    z2d = jnp.zeros((N, HALF), jnp.float32)
    ones2d = jnp.ones((CH, DW), jnp.float32)

    dega, degb = _deg_kernel(dst, ones2d, z2d)

    xw1a, xw1b = pl.pallas_call(
        _tc1_body,
        grid=(NB,),
        in_specs=[_rowspec(F), _fullspec((F, H)), _rowspec(DW), _rowspec(DW)],
        out_specs=[_rowspec(HALF), _rowspec(HALF)],
        out_shape=[jax.ShapeDtypeStruct((N, HALF), jnp.float32)] * 2,
    )(x, W1, dega, degb)

    s1a, s1b = _agg_kernel(xw1a, xw1b, src, dst, z2d)

    xw2a, xw2b = pl.pallas_call(
        _tc2_body,
        grid=(NB,),
        in_specs=[_rowspec(HALF)] * 4 + [_rowspec(DW)] * 2
        + [_fullspec((1, H)), _fullspec((H, H))],
        out_specs=[_rowspec(HALF), _rowspec(HALF)],
        out_shape=[jax.ShapeDtypeStruct((N, HALF), jnp.float32)] * 2,
    )(s1a, s1b, xw1a, xw1b, dega, degb, b1.reshape(1, H), W2)

    s2a, s2b = _agg_kernel(xw2a, xw2b, src, dst, z2d)

    batch3 = batch.reshape(NB, 1, RB)
    _, _, out = pl.pallas_call(
        _tc3_body,
        grid=(NB,),
        in_specs=[_rowspec(HALF)] * 4 + [_rowspec(DW)] * 2
        + [_fullspec((1, H)), pl.BlockSpec((1, 1, RB), lambda i: (i, 0, 0)),
           _fullspec((H, G)), _fullspec((1, G))],
        out_specs=[_fullspec((B, H)), _fullspec((B, G)), _fullspec((B, G))],
        out_shape=[jax.ShapeDtypeStruct((B, H), jnp.float32),
                   jax.ShapeDtypeStruct((B, G), jnp.float32),
                   jax.ShapeDtypeStruct((B, G), jnp.float32)],
    )(s2a, s2b, xw2a, xw2b, dega, degb, b2.reshape(1, H), batch3,
      Wfc, bfc.reshape(1, G))
    return out
